# initial kernel scaffold (unmeasured)
import jax
import jax.numpy as jnp
from jax import lax
from jax.experimental import pallas as pl
from jax.experimental.pallas import tpu as pltpu

N_DEV = 16
N_TOK = 2048
D_IN = 512
D_OUT = 1024
E_LOCAL = 8
ROWS = N_TOK // N_DEV
N_EXPERTS = 128


def kernel(x, router_W, route_idx, expert_W):
    def body(x_ref, rw_ref, idx_ref, ew_ref, out_ref,
             gates_ref, partial_ref, comm_ref, send_buf,
             send_sem, recv_sems):
        my = lax.axis_index("i")
        right = lax.rem(my + 1, N_DEV)
        left = lax.rem(my + N_DEV - 1, N_DEV)

        xv = x_ref[:, :]
        scores = jnp.dot(xv, rw_ref[:, :], preferred_element_type=jnp.float32)
        m = jnp.max(scores, axis=-1, keepdims=True)
        p = jnp.exp(scores - m)
        p = p / jnp.sum(p, axis=-1, keepdims=True)
        eid = lax.broadcasted_iota(jnp.int32, (N_TOK, N_EXPERTS), 1)
        i0 = idx_ref[:, 0:1]
        i1 = idx_ref[:, 1:2]
        mask = jnp.logical_or(eid == i0, eid == i1)
        pm = jnp.where(mask, p, 0.0)
        denom = jnp.sum(pm, axis=-1, keepdims=True)
        gfull = pm / denom

        rows_i = lax.broadcasted_iota(jnp.int32, (N_EXPERTS, E_LOCAL), 0)
        cols_i = lax.broadcasted_iota(jnp.int32, (N_EXPERTS, E_LOCAL), 1)
        sel = (rows_i == my * E_LOCAL + cols_i).astype(jnp.float32)
        gates_ref[:, :] = jnp.dot(gfull, sel, preferred_element_type=jnp.float32)

        acc = jnp.zeros((N_TOK, D_OUT), jnp.float32)
        for j in range(E_LOCAL):
            xg = xv * gates_ref[:, j:j + 1]
            acc = acc + jnp.dot(xg, ew_ref[j], preferred_element_type=jnp.float32)
        partial_ref[:, :] = acc

        for s in range(N_DEV):
            c = lax.rem(my + 2 * N_DEV - 1 - s, N_DEV)
            chunk = partial_ref[pl.ds(c * ROWS, ROWS), :]
            if s > 0:
                recv = pltpu.make_async_remote_copy(
                    src_ref=send_buf,
                    dst_ref=comm_ref.at[s - 1],
                    send_sem=send_sem,
                    recv_sem=recv_sems.at[s - 1],
                    device_id=(left,),
                    device_id_type=pl.DeviceIdType.MESH,
                )
                recv.wait_recv()
                chunk = chunk + comm_ref[s - 1]
            if s < N_DEV - 1:
                send_buf[:, :] = chunk
                rdma = pltpu.make_async_remote_copy(
                    src_ref=send_buf,
                    dst_ref=comm_ref.at[s],
                    send_sem=send_sem,
                    recv_sem=recv_sems.at[s],
                    device_id=(right,),
                    device_id_type=pl.DeviceIdType.MESH,
                )
                rdma.start()
                rdma.wait_send()
            else:
                out_ref[:, :] = chunk

    return pl.pallas_call(
        body,
        out_shape=jax.ShapeDtypeStruct((ROWS, D_OUT), jnp.float32),
        in_specs=[
            pl.BlockSpec(memory_space=pltpu.VMEM),
            pl.BlockSpec(memory_space=pltpu.VMEM),
            pl.BlockSpec(memory_space=pltpu.VMEM),
            pl.BlockSpec(memory_space=pltpu.VMEM),
        ],
        out_specs=pl.BlockSpec(memory_space=pltpu.VMEM),
        scratch_shapes=[
            pltpu.VMEM((N_TOK, E_LOCAL), jnp.float32),
            pltpu.VMEM((N_TOK, D_OUT), jnp.float32),
            pltpu.VMEM((N_DEV - 1, ROWS, D_OUT), jnp.float32),
            pltpu.VMEM((ROWS, D_OUT), jnp.float32),
            pltpu.SemaphoreType.DMA,
            pltpu.SemaphoreType.DMA((N_DEV - 1,)),
        ],
        compiler_params=pltpu.CompilerParams(collective_id=0),
    )(x, router_W, route_idx, expert_W)


# baseline (device time: 155145 ns/iter reference)
import jax
import jax.numpy as jnp
from jax import lax
from jax.experimental import pallas as pl
from jax.experimental.pallas import tpu as pltpu

N_DEV = 16
N_TOK = 2048
D_IN = 512
D_OUT = 1024
E_LOCAL = 8
ROWS = N_TOK // N_DEV
N_EXPERTS = 128


def kernel(x, router_W, route_idx, expert_W):
    def body(x_ref, rw_ref, idx_ref, ew_ref, out_ref,
             gates_ref, partial_ref, comm_ref, send_buf,
             send_sem, recv_sems):
        my = lax.axis_index("i")
        right = lax.rem(my + 1, N_DEV)
        left = lax.rem(my + N_DEV - 1, N_DEV)

        xv = x_ref[:, :]
        scores = jnp.dot(xv, rw_ref[:, :], preferred_element_type=jnp.float32)
        m = jnp.max(scores, axis=-1, keepdims=True)
        p = jnp.exp(scores - m)
        p = p / jnp.sum(p, axis=-1, keepdims=True)
        eid = lax.broadcasted_iota(jnp.int32, (N_TOK, N_EXPERTS), 1)
        i0 = idx_ref[:, 0:1]
        i1 = idx_ref[:, 1:2]
        mask = jnp.logical_or(eid == i0, eid == i1)
        pm = jnp.where(mask, p, 0.0)
        denom = jnp.sum(pm, axis=-1, keepdims=True)
        gfull = pm / denom

        rows_i = lax.broadcasted_iota(jnp.int32, (N_EXPERTS, E_LOCAL), 0)
        cols_i = lax.broadcasted_iota(jnp.int32, (N_EXPERTS, E_LOCAL), 1)
        sel = (rows_i == my * E_LOCAL + cols_i).astype(jnp.float32)
        gates_ref[:, :] = jnp.dot(gfull, sel, preferred_element_type=jnp.float32)

        acc = jnp.zeros((N_TOK, D_OUT), jnp.float32)
        for j in range(E_LOCAL):
            xg = xv * gates_ref[:, j:j + 1]
            acc = acc + jnp.dot(xg, ew_ref[j], preferred_element_type=jnp.float32)
        partial_ref[:, :] = acc

        for s in range(N_DEV):
            c = lax.rem(my + 2 * N_DEV - 1 - s, N_DEV)
            chunk = partial_ref[pl.ds(c * ROWS, ROWS), :]
            if s > 0:
                recv = pltpu.make_async_remote_copy(
                    src_ref=send_buf,
                    dst_ref=comm_ref.at[s - 1],
                    send_sem=send_sem,
                    recv_sem=recv_sems.at[s - 1],
                    device_id=(left,),
                    device_id_type=pl.DeviceIdType.MESH,
                )
                recv.wait_recv()
                chunk = chunk + comm_ref[s - 1]
            if s < N_DEV - 1:
                send_buf[:, :] = chunk
                rdma = pltpu.make_async_remote_copy(
                    src_ref=send_buf,
                    dst_ref=comm_ref.at[s],
                    send_sem=send_sem,
                    recv_sem=recv_sems.at[s],
                    device_id=(right,),
                    device_id_type=pl.DeviceIdType.MESH,
                )
                rdma.start()
                rdma.wait_send()
            else:
                out_ref[:, :] = chunk

    return pl.pallas_call(
        body,
        out_shape=jax.ShapeDtypeStruct((ROWS, D_OUT), jnp.float32),
        in_specs=[
            pl.BlockSpec(memory_space=pltpu.VMEM),
            pl.BlockSpec(memory_space=pltpu.VMEM),
            pl.BlockSpec(memory_space=pltpu.VMEM),
            pl.BlockSpec(memory_space=pltpu.VMEM),
        ],
        out_specs=pl.BlockSpec(memory_space=pltpu.VMEM),
        scratch_shapes=[
            pltpu.VMEM((N_TOK, E_LOCAL), jnp.float32),
            pltpu.VMEM((N_TOK, D_OUT), jnp.float32),
            pltpu.VMEM((N_DEV - 1, ROWS, D_OUT), jnp.float32),
            pltpu.VMEM((ROWS, D_OUT), jnp.float32),
            pltpu.SemaphoreType.DMA,
            pltpu.SemaphoreType.DMA((N_DEV - 1,)),
        ],
        compiler_params=pltpu.CompilerParams(
            vmem_limit_bytes=96 * 1024 * 1024,
        ),
    )(x, router_W, route_idx, expert_W)


# device time: 117155 ns/iter; 1.3243x vs baseline; 1.3243x over previous
import jax
import jax.numpy as jnp
from jax import lax
from jax.experimental import pallas as pl
from jax.experimental.pallas import tpu as pltpu

N_DEV = 16
N_TOK = 2048
D_IN = 512
D_OUT = 1024
E_LOCAL = 8
ROWS = N_TOK // N_DEV
HALF = ROWS // 2
N_EXPERTS = 128


def kernel(x, router_W, route_idx, expert_W):
    def body(x_ref, rw_ref, idx_ref, ew_ref, out_ref,
             gates_ref, comm_r, comm_l, send_r, send_l,
             ssem_r, ssem_l, rsem_r, rsem_l):
        my = lax.axis_index("i")
        right = lax.rem(my + 1, N_DEV)
        left = lax.rem(my + N_DEV - 1, N_DEV)

        xv = x_ref[:, :]
        scores = jnp.dot(xv, rw_ref[:, :], preferred_element_type=jnp.float32)
        m = jnp.max(scores, axis=-1, keepdims=True)
        p = jnp.exp(scores - m)
        p = p / jnp.sum(p, axis=-1, keepdims=True)
        eid = lax.broadcasted_iota(jnp.int32, (N_TOK, N_EXPERTS), 1)
        i0 = idx_ref[:, 0:1]
        i1 = idx_ref[:, 1:2]
        mask = jnp.logical_or(eid == i0, eid == i1)
        pm = jnp.where(mask, p, 0.0)
        denom = jnp.sum(pm, axis=-1, keepdims=True)
        gfull = pm / denom

        rows_i = lax.broadcasted_iota(jnp.int32, (N_EXPERTS, E_LOCAL), 0)
        cols_i = lax.broadcasted_iota(jnp.int32, (N_EXPERTS, E_LOCAL), 1)
        sel = (rows_i == my * E_LOCAL + cols_i).astype(jnp.float32)
        gates_ref[:, :] = jnp.dot(gfull, sel, preferred_element_type=jnp.float32)

        sends_r = []
        sends_l = []
        for s in range(N_DEV):
            c_r = lax.rem(my + 2 * N_DEV - 1 - s, N_DEV)
            c_l = lax.rem(my + 1 + s, N_DEV)
            xc = jnp.concatenate(
                [x_ref[pl.ds(c_r * ROWS, HALF), :],
                 x_ref[pl.ds(c_l * ROWS + HALF, HALF), :]], axis=0)
            gc = jnp.concatenate(
                [gates_ref[pl.ds(c_r * ROWS, HALF), :],
                 gates_ref[pl.ds(c_l * ROWS + HALF, HALF), :]], axis=0)
            acc = jnp.zeros((ROWS, D_OUT), jnp.float32)
            for j in range(E_LOCAL):
                acc = acc + jnp.dot(xc * gc[:, j:j + 1], ew_ref[j],
                                    preferred_element_type=jnp.float32)
            acc_r = acc[:HALF, :]
            acc_l = acc[HALF:, :]
            if s > 0:
                recv_r = pltpu.make_async_remote_copy(
                    src_ref=send_r.at[0],
                    dst_ref=comm_r.at[s - 1],
                    send_sem=ssem_r.at[0],
                    recv_sem=rsem_r.at[s - 1],
                    device_id=(left,),
                    device_id_type=pl.DeviceIdType.MESH,
                )
                recv_r.wait_recv()
                acc_r = acc_r + comm_r[s - 1]
                recv_l = pltpu.make_async_remote_copy(
                    src_ref=send_l.at[0],
                    dst_ref=comm_l.at[s - 1],
                    send_sem=ssem_l.at[0],
                    recv_sem=rsem_l.at[s - 1],
                    device_id=(right,),
                    device_id_type=pl.DeviceIdType.MESH,
                )
                recv_l.wait_recv()
                acc_l = acc_l + comm_l[s - 1]
            if s < N_DEV - 1:
                slot = s % 2
                if s >= 2:
                    sends_r[s - 2].wait_send()
                    sends_l[s - 2].wait_send()
                send_r[slot, :, :] = acc_r
                send_l[slot, :, :] = acc_l
                rdma_r = pltpu.make_async_remote_copy(
                    src_ref=send_r.at[slot],
                    dst_ref=comm_r.at[s],
                    send_sem=ssem_r.at[slot],
                    recv_sem=rsem_r.at[s],
                    device_id=(right,),
                    device_id_type=pl.DeviceIdType.MESH,
                )
                rdma_r.start()
                sends_r.append(rdma_r)
                rdma_l = pltpu.make_async_remote_copy(
                    src_ref=send_l.at[slot],
                    dst_ref=comm_l.at[s],
                    send_sem=ssem_l.at[slot],
                    recv_sem=rsem_l.at[s],
                    device_id=(left,),
                    device_id_type=pl.DeviceIdType.MESH,
                )
                rdma_l.start()
                sends_l.append(rdma_l)
            else:
                out_ref[:HALF, :] = acc_r
                out_ref[HALF:, :] = acc_l
        for d in range(N_DEV - 3, N_DEV - 1):
            sends_r[d].wait_send()
            sends_l[d].wait_send()

    return pl.pallas_call(
        body,
        out_shape=jax.ShapeDtypeStruct((ROWS, D_OUT), jnp.float32),
        in_specs=[
            pl.BlockSpec(memory_space=pltpu.VMEM),
            pl.BlockSpec(memory_space=pltpu.VMEM),
            pl.BlockSpec(memory_space=pltpu.VMEM),
            pl.BlockSpec(memory_space=pltpu.VMEM),
        ],
        out_specs=pl.BlockSpec(memory_space=pltpu.VMEM),
        scratch_shapes=[
            pltpu.VMEM((N_TOK, E_LOCAL), jnp.float32),
            pltpu.VMEM((N_DEV - 1, HALF, D_OUT), jnp.float32),
            pltpu.VMEM((N_DEV - 1, HALF, D_OUT), jnp.float32),
            pltpu.VMEM((2, HALF, D_OUT), jnp.float32),
            pltpu.VMEM((2, HALF, D_OUT), jnp.float32),
            pltpu.SemaphoreType.DMA((2,)),
            pltpu.SemaphoreType.DMA((2,)),
            pltpu.SemaphoreType.DMA((N_DEV - 1,)),
            pltpu.SemaphoreType.DMA((N_DEV - 1,)),
        ],
        compiler_params=pltpu.CompilerParams(
            vmem_limit_bytes=96 * 1024 * 1024,
        ),
    )(x, router_W, route_idx, expert_W)


# device time: 95580 ns/iter; 1.6232x vs baseline; 1.2257x over previous
import jax
import jax.numpy as jnp
from jax import lax
from jax.experimental import pallas as pl
from jax.experimental.pallas import tpu as pltpu

N_DEV = 16
N_TOK = 2048
D_IN = 512
D_OUT = 1024
E_LOCAL = 8
ROWS = N_TOK // N_DEV
HALF = ROWS // 2
N_EXPERTS = 128


def kernel(x, router_W, route_idx, expert_W):
    def body(x_ref, rw_ref, idx_ref, ew_ref, out_ref,
             gates_ref, comm_r, comm_l, send_r, send_l,
             ssem_r, ssem_l, rsem_r, rsem_l):
        my = lax.axis_index("i")
        right = lax.rem(my + 1, N_DEV)
        left = lax.rem(my + N_DEV - 1, N_DEV)

        xv = x_ref[:, :]
        scores = jnp.dot(xv, rw_ref[:, :], preferred_element_type=jnp.float32)
        m = jnp.max(scores, axis=-1, keepdims=True)
        p = jnp.exp(scores - m)
        p = p / jnp.sum(p, axis=-1, keepdims=True)
        eid = lax.broadcasted_iota(jnp.int32, (N_TOK, N_EXPERTS), 1)
        i0 = idx_ref[:, 0:1]
        i1 = idx_ref[:, 1:2]
        mask = jnp.logical_or(eid == i0, eid == i1)
        pm = jnp.where(mask, p, 0.0)
        denom = jnp.sum(pm, axis=-1, keepdims=True)
        gfull = pm / denom

        rows_i = lax.broadcasted_iota(jnp.int32, (N_EXPERTS, E_LOCAL), 0)
        cols_i = lax.broadcasted_iota(jnp.int32, (N_EXPERTS, E_LOCAL), 1)
        sel = (rows_i == my * E_LOCAL + cols_i).astype(jnp.float32)
        gates_ref[:, :] = jnp.dot(gfull, sel, preferred_element_type=jnp.float32)

        def compute_partial(s):
            c_r = lax.rem(my + 2 * N_DEV - 1 - s, N_DEV)
            c_l = lax.rem(my + 1 + s, N_DEV)
            xc = jnp.concatenate(
                [x_ref[pl.ds(c_r * ROWS, HALF), :],
                 x_ref[pl.ds(c_l * ROWS + HALF, HALF), :]], axis=0)
            gc = jnp.concatenate(
                [gates_ref[pl.ds(c_r * ROWS, HALF), :],
                 gates_ref[pl.ds(c_l * ROWS + HALF, HALF), :]], axis=0)
            acc = jnp.zeros((ROWS, D_OUT), jnp.float32)
            for j in range(E_LOCAL):
                acc = acc + jnp.dot(xc * gc[:, j:j + 1], ew_ref[j],
                                    preferred_element_type=jnp.float32)
            return acc

        sends_r = []
        sends_l = []
        p = compute_partial(0)
        for s in range(N_DEV):
            acc_r = p[:HALF, :]
            acc_l = p[HALF:, :]
            if s > 0:
                recv_r = pltpu.make_async_remote_copy(
                    src_ref=send_r.at[0],
                    dst_ref=comm_r.at[s - 1],
                    send_sem=ssem_r.at[0],
                    recv_sem=rsem_r.at[s - 1],
                    device_id=(left,),
                    device_id_type=pl.DeviceIdType.MESH,
                )
                recv_r.wait_recv()
                acc_r = acc_r + comm_r[s - 1].astype(jnp.float32)
                recv_l = pltpu.make_async_remote_copy(
                    src_ref=send_l.at[0],
                    dst_ref=comm_l.at[s - 1],
                    send_sem=ssem_l.at[0],
                    recv_sem=rsem_l.at[s - 1],
                    device_id=(right,),
                    device_id_type=pl.DeviceIdType.MESH,
                )
                recv_l.wait_recv()
                acc_l = acc_l + comm_l[s - 1].astype(jnp.float32)
            if s < N_DEV - 1:
                slot = s % 2
                if s >= 2:
                    sends_r[s - 2].wait_send()
                    sends_l[s - 2].wait_send()
                send_r[slot, :, :] = acc_r.astype(jnp.bfloat16)
                send_l[slot, :, :] = acc_l.astype(jnp.bfloat16)
                rdma_r = pltpu.make_async_remote_copy(
                    src_ref=send_r.at[slot],
                    dst_ref=comm_r.at[s],
                    send_sem=ssem_r.at[slot],
                    recv_sem=rsem_r.at[s],
                    device_id=(right,),
                    device_id_type=pl.DeviceIdType.MESH,
                )
                rdma_r.start()
                sends_r.append(rdma_r)
                rdma_l = pltpu.make_async_remote_copy(
                    src_ref=send_l.at[slot],
                    dst_ref=comm_l.at[s],
                    send_sem=ssem_l.at[slot],
                    recv_sem=rsem_l.at[s],
                    device_id=(left,),
                    device_id_type=pl.DeviceIdType.MESH,
                )
                rdma_l.start()
                sends_l.append(rdma_l)
                p = compute_partial(s + 1)
            else:
                out_ref[:HALF, :] = acc_r
                out_ref[HALF:, :] = acc_l
        for d in range(N_DEV - 3, N_DEV - 1):
            sends_r[d].wait_send()
            sends_l[d].wait_send()

    return pl.pallas_call(
        body,
        out_shape=jax.ShapeDtypeStruct((ROWS, D_OUT), jnp.float32),
        in_specs=[
            pl.BlockSpec(memory_space=pltpu.VMEM),
            pl.BlockSpec(memory_space=pltpu.VMEM),
            pl.BlockSpec(memory_space=pltpu.VMEM),
            pl.BlockSpec(memory_space=pltpu.VMEM),
        ],
        out_specs=pl.BlockSpec(memory_space=pltpu.VMEM),
        scratch_shapes=[
            pltpu.VMEM((N_TOK, E_LOCAL), jnp.float32),
            pltpu.VMEM((N_DEV - 1, HALF, D_OUT), jnp.bfloat16),
            pltpu.VMEM((N_DEV - 1, HALF, D_OUT), jnp.bfloat16),
            pltpu.VMEM((2, HALF, D_OUT), jnp.bfloat16),
            pltpu.VMEM((2, HALF, D_OUT), jnp.bfloat16),
            pltpu.SemaphoreType.DMA((2,)),
            pltpu.SemaphoreType.DMA((2,)),
            pltpu.SemaphoreType.DMA((N_DEV - 1,)),
            pltpu.SemaphoreType.DMA((N_DEV - 1,)),
        ],
        compiler_params=pltpu.CompilerParams(
            vmem_limit_bytes=96 * 1024 * 1024,
        ),
    )(x, router_W, route_idx, expert_W)


# device time: 75376 ns/iter; 2.0583x vs baseline; 1.2680x over previous
import jax
import jax.numpy as jnp
from jax import lax
from jax.experimental import pallas as pl
from jax.experimental.pallas import tpu as pltpu

N_DEV = 16
N_PLANE = 4
N_Z = 4
N_TOK = 2048
D_IN = 512
D_OUT = 1024
E_LOCAL = 8
ROWS = N_TOK // N_DEV
HALF = ROWS // 2
GROUP = N_Z * HALF
N_EXPERTS = 128


def kernel(x, router_W, route_idx, expert_W):
    def body(x_ref, rw_ref, idx_ref, ew_ref, out_ref,
             gates_ref, gacc_ref,
             comm_ar, comm_al, send_ar, send_al,
             comm_br, comm_bl, send_br, send_bl,
             ssem_ar, ssem_al, ssem_br, ssem_bl,
             rsem_ar, rsem_al, rsem_br, rsem_bl):
        my = lax.axis_index("i")
        z = my // N_PLANE
        q = lax.rem(my, N_PLANE)
        a_right = z * N_PLANE + lax.rem(q + 1, N_PLANE)
        a_left = z * N_PLANE + lax.rem(q + 3, N_PLANE)
        b_right = lax.rem(my + N_PLANE, N_DEV)
        b_left = lax.rem(my + N_DEV - N_PLANE, N_DEV)

        xv = x_ref[:, :]
        scores = jnp.dot(xv, rw_ref[:, :], preferred_element_type=jnp.float32)
        m = jnp.max(scores, axis=-1, keepdims=True)
        p = jnp.exp(scores - m)
        p = p / jnp.sum(p, axis=-1, keepdims=True)
        eid = lax.broadcasted_iota(jnp.int32, (N_TOK, N_EXPERTS), 1)
        mask = jnp.logical_or(eid == idx_ref[:, 0:1], eid == idx_ref[:, 1:2])
        pm = jnp.where(mask, p, 0.0)
        gfull = pm / jnp.sum(pm, axis=-1, keepdims=True)
        rows_i = lax.broadcasted_iota(jnp.int32, (N_EXPERTS, E_LOCAL), 0)
        cols_i = lax.broadcasted_iota(jnp.int32, (N_EXPERTS, E_LOCAL), 1)
        sel = (rows_i == my * E_LOCAL + cols_i).astype(jnp.float32)
        gates_ref[:, :] = jnp.dot(gfull, sel, preferred_element_type=jnp.float32)

        def make_recv(dst, rsem, dummy_src, dummy_ssem):
            return pltpu.make_async_remote_copy(
                src_ref=dummy_src, dst_ref=dst,
                send_sem=dummy_ssem, recv_sem=rsem,
                device_id=(my,), device_id_type=pl.DeviceIdType.MESH,
            )

        def compute_group(a):
            qr = lax.rem(q + 2 * N_PLANE - 1 - a, N_PLANE)
            ql = lax.rem(q + 1 + a, N_PLANE)
            xs, gs = [], []
            for zp in range(N_Z):
                off = zp * N_PLANE * ROWS + qr * ROWS
                xs.append(x_ref[pl.ds(off, HALF), :])
                gs.append(gates_ref[pl.ds(off, HALF), :])
            for zp in range(N_Z):
                off = zp * N_PLANE * ROWS + ql * ROWS + HALF
                xs.append(x_ref[pl.ds(off, HALF), :])
                gs.append(gates_ref[pl.ds(off, HALF), :])
            xc = jnp.concatenate(xs, axis=0)
            gc = jnp.concatenate(gs, axis=0)
            acc = jnp.zeros((2 * GROUP, D_OUT), jnp.float32)
            for j in range(E_LOCAL):
                acc = acc + jnp.dot(xc * gc[:, j:j + 1], ew_ref[j],
                                    preferred_element_type=jnp.float32)
            return acc

        sends_ar, sends_al = [], []
        pa = compute_group(0)
        for a in range(N_PLANE):
            acc_r = pa[:GROUP, :]
            acc_l = pa[GROUP:, :]
            if a > 0:
                make_recv(comm_ar.at[a - 1], rsem_ar.at[a - 1],
                          send_ar.at[0], ssem_ar.at[0]).wait_recv()
                acc_r = acc_r + comm_ar[a - 1].astype(jnp.float32)
                make_recv(comm_al.at[a - 1], rsem_al.at[a - 1],
                          send_al.at[0], ssem_al.at[0]).wait_recv()
                acc_l = acc_l + comm_al[a - 1].astype(jnp.float32)
            if a < N_PLANE - 1:
                slot = a % 2
                if a >= 2:
                    sends_ar[a - 2].wait_send()
                    sends_al[a - 2].wait_send()
                send_ar[slot, :, :] = acc_r.astype(jnp.bfloat16)
                send_al[slot, :, :] = acc_l.astype(jnp.bfloat16)
                rdma_r = pltpu.make_async_remote_copy(
                    src_ref=send_ar.at[slot], dst_ref=comm_ar.at[a],
                    send_sem=ssem_ar.at[slot], recv_sem=rsem_ar.at[a],
                    device_id=(a_right,), device_id_type=pl.DeviceIdType.MESH,
                )
                rdma_r.start()
                sends_ar.append(rdma_r)
                rdma_l = pltpu.make_async_remote_copy(
                    src_ref=send_al.at[slot], dst_ref=comm_al.at[a],
                    send_sem=ssem_al.at[slot], recv_sem=rsem_al.at[a],
                    device_id=(a_left,), device_id_type=pl.DeviceIdType.MESH,
                )
                rdma_l.start()
                sends_al.append(rdma_l)
                pa = compute_group(a + 1)
            else:
                for zp in range(N_Z):
                    gacc_ref[zp * ROWS: zp * ROWS + HALF, :] = \
                        acc_r[zp * HALF:(zp + 1) * HALF, :]
                    gacc_ref[zp * ROWS + HALF:(zp + 1) * ROWS, :] = \
                        acc_l[zp * HALF:(zp + 1) * HALF, :]

        sends_br, sends_bl = [], []
        for b in range(N_Z):
            zr = lax.rem(z + 2 * N_Z - 1 - b, N_Z)
            zl = lax.rem(z + 1 + b, N_Z)
            acc_r = gacc_ref[pl.ds(zr * ROWS, HALF), :]
            acc_l = gacc_ref[pl.ds(zl * ROWS + HALF, HALF), :]
            if b > 0:
                make_recv(comm_br.at[b - 1], rsem_br.at[b - 1],
                          send_br.at[0], ssem_br.at[0]).wait_recv()
                acc_r = acc_r + comm_br[b - 1].astype(jnp.float32)
                make_recv(comm_bl.at[b - 1], rsem_bl.at[b - 1],
                          send_bl.at[0], ssem_bl.at[0]).wait_recv()
                acc_l = acc_l + comm_bl[b - 1].astype(jnp.float32)
            if b < N_Z - 1:
                slot = b % 2
                if b >= 2:
                    sends_br[b - 2].wait_send()
                    sends_bl[b - 2].wait_send()
                send_br[slot, :, :] = acc_r.astype(jnp.bfloat16)
                send_bl[slot, :, :] = acc_l.astype(jnp.bfloat16)
                rdma_r = pltpu.make_async_remote_copy(
                    src_ref=send_br.at[slot], dst_ref=comm_br.at[b],
                    send_sem=ssem_br.at[slot], recv_sem=rsem_br.at[b],
                    device_id=(b_right,), device_id_type=pl.DeviceIdType.MESH,
                )
                rdma_r.start()
                sends_br.append(rdma_r)
                rdma_l = pltpu.make_async_remote_copy(
                    src_ref=send_bl.at[slot], dst_ref=comm_bl.at[b],
                    send_sem=ssem_bl.at[slot], recv_sem=rsem_bl.at[b],
                    device_id=(b_left,), device_id_type=pl.DeviceIdType.MESH,
                )
                rdma_l.start()
                sends_bl.append(rdma_l)
            else:
                out_ref[:HALF, :] = acc_r
                out_ref[HALF:, :] = acc_l
        for lst in (sends_ar, sends_al, sends_br, sends_bl):
            for d in (1, 2):
                lst[d].wait_send()

    bf = jnp.bfloat16
    return pl.pallas_call(
        body,
        out_shape=jax.ShapeDtypeStruct((ROWS, D_OUT), jnp.float32),
        in_specs=[pl.BlockSpec(memory_space=pltpu.VMEM)] * 4,
        out_specs=pl.BlockSpec(memory_space=pltpu.VMEM),
        scratch_shapes=[
            pltpu.VMEM((N_TOK, E_LOCAL), jnp.float32),
            pltpu.VMEM((N_Z * ROWS, D_OUT), jnp.float32),
            pltpu.VMEM((N_PLANE - 1, GROUP, D_OUT), bf),
            pltpu.VMEM((N_PLANE - 1, GROUP, D_OUT), bf),
            pltpu.VMEM((2, GROUP, D_OUT), bf),
            pltpu.VMEM((2, GROUP, D_OUT), bf),
            pltpu.VMEM((N_Z - 1, HALF, D_OUT), bf),
            pltpu.VMEM((N_Z - 1, HALF, D_OUT), bf),
            pltpu.VMEM((2, HALF, D_OUT), bf),
            pltpu.VMEM((2, HALF, D_OUT), bf),
            pltpu.SemaphoreType.DMA((2,)),
            pltpu.SemaphoreType.DMA((2,)),
            pltpu.SemaphoreType.DMA((2,)),
            pltpu.SemaphoreType.DMA((2,)),
            pltpu.SemaphoreType.DMA((N_PLANE - 1,)),
            pltpu.SemaphoreType.DMA((N_PLANE - 1,)),
            pltpu.SemaphoreType.DMA((N_Z - 1,)),
            pltpu.SemaphoreType.DMA((N_Z - 1,)),
        ],
        compiler_params=pltpu.CompilerParams(
            vmem_limit_bytes=96 * 1024 * 1024,
        ),
    )(x, router_W, route_idx, expert_W)
